# baseline (speedup 1.0000x reference)
import functools

import numpy as np
import jax
import jax.numpy as jnp
from jax import lax
from jax.experimental import pallas as pl
from jax.experimental.pallas import tpu as pltpu
from jax.experimental.pallas import tpu_sc as plsc

DEG = 3
NCP = 64
NSEG = NCP - DEG
L = 16

_KNOT_N = np.float32(NSEG - 1) / np.float32(NSEG)
_CLOSE_THR = np.float32(1e-5 + 1e-5 * float(_KNOT_N))

_RCP_COMBOS = ((1, 0), (1, -1), (2, 0), (1, -2), (2, -1), (3, 0))
_RCP_BASE = 72


def _build_table():
    tab = []
    gg = np.arange(64)
    for j in (-2, -1, 0, 1, 2, 3):
        tab.append((np.clip(gg + j, 0, NSEG).astype(np.float64) / NSEG
                    ).astype(np.float32))
    ci = lambda i: np.clip(gg + i, 0, NSEG)
    for hi, lo in _RCP_COMBOS:
        d = np.maximum(ci(hi) - ci(lo), 1)
        tab.append((np.float32(NSEG) / d.astype(np.float32)).astype(np.float32))
    return np.stack(tab).reshape(-1)


_TABLE = _build_table()


def _f(v):
    return jnp.full((L,), v, dtype=jnp.float32)


def _i(v):
    return jnp.full((L,), v, dtype=jnp.int32)


def _span(u):
    g = (u * _f(NSEG)).astype(jnp.int32)
    g = jnp.minimum(jnp.maximum(g, _i(0)), _i(NSEG - 1))
    close = jnp.abs(u - _f(_KNOT_N)) <= _f(_CLOSE_THR)
    return jnp.where(close, _i(NCP - 1 - DEG), g)


def _basis(u, g, tab_gather):
    tf = {j: tab_gather(k)(g) for k, j in enumerate((-2, -1, 0, 1, 2, 3))}
    rc = {c: tab_gather(6 + k)(g) for k, c in enumerate(_RCP_COMBOS)}
    left = {j: u - tf[1 - j] for j in (1, 2, 3)}
    right = {j: tf[j] - u for j in (1, 2, 3)}

    r10 = rc[(1, 0)]
    ndu01 = right[1] * r10
    ndu11 = left[1] * r10
    tmp = ndu01 * rc[(1, -1)]
    ndu02 = right[1] * tmp
    saved = left[2] * tmp
    tmp = ndu11 * rc[(2, 0)]
    ndu12 = saved + right[2] * tmp
    ndu22 = left[1] * tmp
    p0 = ndu02 * rc[(1, -2)]
    b0 = right[1] * p0
    saved = left[3] * p0
    p1 = ndu12 * rc[(2, -1)]
    b1 = saved + right[2] * p1
    saved = left[2] * p1
    p2 = ndu22 * rc[(3, 0)]
    b2 = saved + right[3] * p2
    b3 = left[1] * p2
    deg = _f(float(DEG))
    d0 = -(deg * p0)
    d1 = deg * (p0 - p1)
    d2 = deg * (p1 - p2)
    d3 = deg * p2
    return (b0, b1, b2, b3), (d0, d1, d2, d3)


def _rsqrt(x):
    i = jax.lax.bitcast_convert_type(x, jnp.int32)
    i = _i(0x5F3759DF) - jax.lax.shift_right_logical(i, _i(1))
    y = jax.lax.bitcast_convert_type(i, jnp.float32)
    half = x * _f(0.5)
    for _ in range(2):
        y = y * (_f(1.5) - half * y * y)
    return y


def _make_sc_call(n_eval):
    info = plsc.get_sparse_core_info()
    nw = info.num_cores * info.num_subcores
    per_tile = n_eval // nw
    ncp2 = NCP * NCP
    ntab = _TABLE.shape[0]

    mesh = plsc.VectorSubcoreMesh(core_axis_name="c", subcore_axis_name="s")

    @functools.partial(
        pl.kernel,
        out_type=(
            jax.ShapeDtypeStruct((4 * n_eval,), jnp.float32),
            jax.ShapeDtypeStruct((3 * n_eval,), jnp.float32),
        ),
        mesh=mesh,
        compiler_params=pltpu.CompilerParams(needs_layout_passes=False),
        scratch_types=[
            pltpu.VMEM((per_tile,), jnp.float32),
            pltpu.VMEM((per_tile,), jnp.float32),
            pltpu.VMEM((3 * ncp2,), jnp.float32),
            [pltpu.VMEM((64,), jnp.float32) for _ in range(12)],
            [pltpu.VMEM((per_tile,), jnp.float32) for _ in range(4)],
            [pltpu.VMEM((per_tile,), jnp.float32) for _ in range(3)],
            pltpu.SemaphoreType.DMA,
        ],
    )
    def nurbs_sc(x_hbm, y_hbm, cp_hbm, tab_hbm, outs_hbm, outn_hbm,
                 x_v, y_v, cp_v, tab_v, os_v, on_v, sem):
        wid = lax.axis_index("s") * info.num_cores + lax.axis_index("c")
        base = wid * per_tile
        copies = [
            pltpu.async_copy(x_hbm.at[pl.ds(base, per_tile)], x_v, sem),
            pltpu.async_copy(y_hbm.at[pl.ds(base, per_tile)], y_v, sem),
            pltpu.async_copy(cp_hbm.at[pl.ds(0, 3 * ncp2)], cp_v, sem),
        ] + [
            pltpu.async_copy(tab_hbm.at[pl.ds(64 * k, 64)], tab_v[k], sem)
            for k in range(12)
        ] + [
        ]
        for cp in copies:
            cp.wait()

        @plsc.parallel_loop(0, per_tile, step=L, unroll=5)
        def body(off):
            u = x_v[pl.ds(off, L)]
            v = y_v[pl.ds(off, L)]
            gx = _span(u)
            gy = _span(v)
            tab_gather = lambda k: (lambda idx: plsc.load_gather(tab_v[k], [idx]))
            (bx0, bx1) = _basis(u, gx, tab_gather)
            (by0, by1) = _basis(v, gy, tab_gather)
            flat3 = (gx * _i(3 * NCP)) + (gy * _i(3))

            surf = [None, None, None]
            du = [None, None, None]
            dv = [None, None, None]
            for s4 in range(4):
                ts = [None, None, None]
                td = [None, None, None]
                for r4 in range(4):
                    idx = flat3 + _i(3 * (r4 * NCP + s4))
                    for c in range(3):
                        g = plsc.load_gather(cp_v, [idx + _i(c)])
                        if r4 == 0:
                            ts[c] = bx0[r4] * g
                            td[c] = bx1[r4] * g
                        else:
                            ts[c] = ts[c] + bx0[r4] * g
                            td[c] = td[c] + bx1[r4] * g
                for c in range(3):
                    if s4 == 0:
                        surf[c] = by0[s4] * ts[c]
                        du[c] = by0[s4] * td[c]
                        dv[c] = by1[s4] * ts[c]
                    else:
                        surf[c] = surf[c] + by0[s4] * ts[c]
                        du[c] = du[c] + by0[s4] * td[c]
                        dv[c] = dv[c] + by1[s4] * ts[c]

            w = (((bx0[0] + bx0[1]) + (bx0[2] + bx0[3]))
                 * ((by0[0] + by0[1]) + (by0[2] + by0[3])))

            nx = du[1] * dv[2] - du[2] * dv[1]
            ny = du[2] * dv[0] - du[0] * dv[2]
            nz = du[0] * dv[1] - du[1] * dv[0]
            rn = _rsqrt(nx * nx + ny * ny + nz * nz)

            os_v[0][pl.ds(off, L)] = surf[0]
            os_v[1][pl.ds(off, L)] = surf[1]
            os_v[2][pl.ds(off, L)] = surf[2]
            os_v[3][pl.ds(off, L)] = w
            on_v[0][pl.ds(off, L)] = nx * rn
            on_v[1][pl.ds(off, L)] = ny * rn
            on_v[2][pl.ds(off, L)] = nz * rn

        out_copies = [
            pltpu.async_copy(os_v[c], outs_hbm.at[pl.ds(c * n_eval + base, per_tile)], sem)
            for c in range(4)
        ] + [
            pltpu.async_copy(on_v[c], outn_hbm.at[pl.ds(c * n_eval + base, per_tile)], sem)
            for c in range(3)
        ]
        for cp in out_copies:
            cp.wait()

    return nurbs_sc


def kernel(evaluation_points_x, evaluation_points_y, control_points):
    n_eval = evaluation_points_x.shape[0]
    cp_flat = control_points.reshape(-1)
    sc = _make_sc_call(n_eval)
    out_s, out_n = sc(evaluation_points_x, evaluation_points_y, cp_flat,
                      jnp.asarray(_TABLE))
    return (out_s.reshape(4, n_eval).T, out_n.reshape(3, n_eval).T)


# speedup vs baseline: 1.0312x; 1.0312x over previous
import functools

import numpy as np
import jax
import jax.numpy as jnp
from jax import lax
from jax.experimental import pallas as pl
from jax.experimental.pallas import tpu as pltpu
from jax.experimental.pallas import tpu_sc as plsc

DEG = 3
NCP = 64
NSEG = NCP - DEG
L = 16

_KNOT_N = np.float32(NSEG - 1) / np.float32(NSEG)
_CLOSE_THR = np.float32(1e-5 + 1e-5 * float(_KNOT_N))

_RCP_COMBOS = ((1, 0), (1, -1), (2, 0), (1, -2), (2, -1), (3, 0))
_RCP_BASE = 72


def _build_table():
    ks = np.clip(np.arange(72) - 2, 0, NSEG).astype(np.float64) / NSEG
    tab = [ks.astype(np.float32)]
    ci = lambda i: np.clip(np.arange(64) + i, 0, NSEG)
    for hi, lo in _RCP_COMBOS:
        d = np.maximum(ci(hi) - ci(lo), 1)
        tab.append((np.float32(NSEG) / d.astype(np.float32)).astype(np.float32))
    return np.concatenate(tab)


_TABLE = _build_table()


def _f(v):
    return jnp.full((L,), v, dtype=jnp.float32)


def _i(v):
    return jnp.full((L,), v, dtype=jnp.int32)


def _span(u):
    g = (u * _f(NSEG)).astype(jnp.int32)
    g = jnp.minimum(jnp.maximum(g, _i(0)), _i(NSEG - 1))
    close = jnp.abs(u - _f(_KNOT_N)) <= _f(_CLOSE_THR)
    return jnp.where(close, _i(NCP - 1 - DEG), g)


def _basis(u, g, tab_gather):
    tf = {j: tab_gather(g + _i(j + 2)) for j in (-2, -1, 0, 1, 2, 3)}
    rc = {c: tab_gather(g + _i(_RCP_BASE + k * 64))
          for k, c in enumerate(_RCP_COMBOS)}
    left = {j: u - tf[1 - j] for j in (1, 2, 3)}
    right = {j: tf[j] - u for j in (1, 2, 3)}

    r10 = rc[(1, 0)]
    ndu01 = right[1] * r10
    ndu11 = left[1] * r10
    tmp = ndu01 * rc[(1, -1)]
    ndu02 = right[1] * tmp
    saved = left[2] * tmp
    tmp = ndu11 * rc[(2, 0)]
    ndu12 = saved + right[2] * tmp
    ndu22 = left[1] * tmp
    p0 = ndu02 * rc[(1, -2)]
    b0 = right[1] * p0
    saved = left[3] * p0
    p1 = ndu12 * rc[(2, -1)]
    b1 = saved + right[2] * p1
    saved = left[2] * p1
    p2 = ndu22 * rc[(3, 0)]
    b2 = saved + right[3] * p2
    b3 = left[1] * p2
    deg = _f(float(DEG))
    d0 = -(deg * p0)
    d1 = deg * (p0 - p1)
    d2 = deg * (p1 - p2)
    d3 = deg * p2
    return (b0, b1, b2, b3), (d0, d1, d2, d3)


def _rsqrt(x):
    i = jax.lax.bitcast_convert_type(x, jnp.int32)
    i = _i(0x5F3759DF) - jax.lax.shift_right_logical(i, _i(1))
    y = jax.lax.bitcast_convert_type(i, jnp.float32)
    half = x * _f(0.5)
    for _ in range(2):
        y = y * (_f(1.5) - half * y * y)
    return y


def _make_sc_call(n_eval):
    info = plsc.get_sparse_core_info()
    nw = info.num_cores * info.num_subcores
    per_tile = n_eval // nw
    ncp2 = NCP * NCP
    ntab = _TABLE.shape[0]

    mesh = plsc.VectorSubcoreMesh(core_axis_name="c", subcore_axis_name="s")

    @functools.partial(
        pl.kernel,
        out_type=(
            jax.ShapeDtypeStruct((4 * n_eval,), jnp.float32),
            jax.ShapeDtypeStruct((3 * n_eval,), jnp.float32),
        ),
        mesh=mesh,
        compiler_params=pltpu.CompilerParams(needs_layout_passes=False),
        scratch_types=[
            pltpu.VMEM((per_tile,), jnp.float32),
            pltpu.VMEM((per_tile,), jnp.float32),
            pltpu.VMEM((3 * ncp2,), jnp.float32),
            pltpu.VMEM((ntab,), jnp.float32),
            [pltpu.VMEM((per_tile,), jnp.float32) for _ in range(4)],
            [pltpu.VMEM((per_tile,), jnp.float32) for _ in range(3)],
            pltpu.SemaphoreType.DMA,
        ],
    )
    def nurbs_sc(x_hbm, y_hbm, cp_hbm, tab_hbm, outs_hbm, outn_hbm,
                 x_v, y_v, cp_v, tab_v, os_v, on_v, sem):
        wid = lax.axis_index("s") * info.num_cores + lax.axis_index("c")
        base = wid * per_tile
        copies = [
            pltpu.async_copy(x_hbm.at[pl.ds(base, per_tile)], x_v, sem),
            pltpu.async_copy(y_hbm.at[pl.ds(base, per_tile)], y_v, sem),
            pltpu.async_copy(cp_hbm.at[pl.ds(0, 3 * ncp2)], cp_v, sem),
            pltpu.async_copy(tab_hbm.at[pl.ds(0, ntab)], tab_v, sem),
        ]
        for cp in copies:
            cp.wait()

        @plsc.parallel_loop(0, per_tile, step=L, unroll=5)
        def body(off):
            u = x_v[pl.ds(off, L)]
            v = y_v[pl.ds(off, L)]
            gx = _span(u)
            gy = _span(v)
            tab_gather = lambda idx: plsc.load_gather(tab_v, [idx])
            (bx0, bx1) = _basis(u, gx, tab_gather)
            (by0, by1) = _basis(v, gy, tab_gather)
            flat3 = (gx * _i(3 * NCP)) + (gy * _i(3))

            surf = [None, None, None]
            du = [None, None, None]
            dv = [None, None, None]
            for s4 in range(4):
                ts = [None, None, None]
                td = [None, None, None]
                for r4 in range(4):
                    idx = flat3 + _i(3 * (r4 * NCP + s4))
                    for c in range(3):
                        g = plsc.load_gather(cp_v, [idx + _i(c)])
                        if r4 == 0:
                            ts[c] = bx0[r4] * g
                            td[c] = bx1[r4] * g
                        else:
                            ts[c] = ts[c] + bx0[r4] * g
                            td[c] = td[c] + bx1[r4] * g
                for c in range(3):
                    if s4 == 0:
                        surf[c] = by0[s4] * ts[c]
                        du[c] = by0[s4] * td[c]
                        dv[c] = by1[s4] * ts[c]
                    else:
                        surf[c] = surf[c] + by0[s4] * ts[c]
                        du[c] = du[c] + by0[s4] * td[c]
                        dv[c] = dv[c] + by1[s4] * ts[c]

            w = (((bx0[0] + bx0[1]) + (bx0[2] + bx0[3]))
                 * ((by0[0] + by0[1]) + (by0[2] + by0[3])))

            nx = du[1] * dv[2] - du[2] * dv[1]
            ny = du[2] * dv[0] - du[0] * dv[2]
            nz = du[0] * dv[1] - du[1] * dv[0]
            rn = _rsqrt(nx * nx + ny * ny + nz * nz)

            os_v[0][pl.ds(off, L)] = surf[0]
            os_v[1][pl.ds(off, L)] = surf[1]
            os_v[2][pl.ds(off, L)] = surf[2]
            os_v[3][pl.ds(off, L)] = w
            on_v[0][pl.ds(off, L)] = nx * rn
            on_v[1][pl.ds(off, L)] = ny * rn
            on_v[2][pl.ds(off, L)] = nz * rn

        out_copies = [
            pltpu.async_copy(os_v[c], outs_hbm.at[pl.ds(c * n_eval + base, per_tile)], sem)
            for c in range(4)
        ] + [
            pltpu.async_copy(on_v[c], outn_hbm.at[pl.ds(c * n_eval + base, per_tile)], sem)
            for c in range(3)
        ]
        for cp in out_copies:
            cp.wait()

    return nurbs_sc


def kernel(evaluation_points_x, evaluation_points_y, control_points):
    n_eval = evaluation_points_x.shape[0]
    cp_flat = control_points.reshape(-1)
    sc = _make_sc_call(n_eval)
    out_s, out_n = sc(evaluation_points_x, evaluation_points_y, cp_flat,
                      jnp.asarray(_TABLE))
    return (out_s.reshape(4, n_eval).T, out_n.reshape(3, n_eval).T)
